# Initial kernel scaffold; baseline (speedup 1.0000x reference)
import jax
import jax.numpy as jnp
from jax.experimental import pallas as pl


def kernel(output, target, bins):
    raise NotImplementedError("write your pallas kernel here")


# speedup vs baseline: 38.0748x; 38.0748x over previous
import functools

import jax
import jax.numpy as jnp
from jax.experimental import pallas as pl
from jax.experimental.pallas import tpu as pltpu

_NUM_CLASSES = 255
_LOWER = -20.0
_UPPER = 20.0
_BIN_LENGTH = (_UPPER - _LOWER) / (_NUM_CLASSES - 1)
_ROWS_PER_BLOCK = 1024


def _body(out_ref, tgt_ref, bins_ref, sum_ref, cnt_ref):
    step = pl.program_id(0)
    x = out_ref[...]
    r = x.shape[0]
    m = jnp.max(x, axis=1, keepdims=True)
    ls = jnp.log(jnp.sum(jnp.exp(x - m), axis=1, keepdims=True))

    t = tgt_ref[...]
    tl = jnp.sign(t) * jnp.log1p(jnp.abs(t))
    b = bins_ref[...]
    idx = jnp.sum((b < tl).astype(jnp.int32), axis=1, keepdims=True)
    lo = idx - 1
    hi = idx

    j = jax.lax.broadcasted_iota(jnp.int32, (r, _NUM_CLASSES), 1)
    lo_wrap = jnp.where(lo < 0, lo + _NUM_CLASSES, lo)
    bin_lo = jnp.sum(jnp.where(j == lo_wrap, b, 0.0), axis=1, keepdims=True)
    w = jnp.clip((tl - bin_lo) / _BIN_LENGTH, 0.0, 1.0)

    a_lo = jnp.where(lo >= 0, 1.0 - w, 0.0)
    a_hi = jnp.where(hi <= _NUM_CLASSES - 1, w, 0.0)

    lo_c = jnp.clip(lo, 0, _NUM_CLASSES - 1)
    hi_c = jnp.clip(hi, 0, _NUM_CLASSES - 1)
    o_lo = jnp.sum(jnp.where(j == lo_c, x, 0.0), axis=1, keepdims=True)
    o_hi = jnp.sum(jnp.where(j == hi_c, x, 0.0), axis=1, keepdims=True)

    loss = -(a_lo * ((o_lo - m) - ls) + a_hi * ((o_hi - m) - ls))

    psum = jnp.sum(loss)
    pcnt = jnp.sum((loss != 0.0).astype(jnp.float32))

    @pl.when(step == 0)
    def _init():
        sum_ref[0, 0] = 0.0
        cnt_ref[0, 0] = 0.0

    sum_ref[0, 0] += psum
    cnt_ref[0, 0] += pcnt


@jax.jit
def kernel(output, target, bins):
    n, c = output.shape
    r = _ROWS_PER_BLOCK
    grid = (n // r,)
    ssum, cnt = pl.pallas_call(
        _body,
        grid=grid,
        in_specs=[
            pl.BlockSpec((r, c), lambda i: (i, 0)),
            pl.BlockSpec((r, 1), lambda i: (i, 0)),
            pl.BlockSpec((1, c), lambda i: (0, 0)),
        ],
        out_specs=[
            pl.BlockSpec(memory_space=pltpu.SMEM),
            pl.BlockSpec(memory_space=pltpu.SMEM),
        ],
        out_shape=[
            jax.ShapeDtypeStruct((1, 1), jnp.float32),
            jax.ShapeDtypeStruct((1, 1), jnp.float32),
        ],
    )(output, target.reshape(n, 1), bins.reshape(1, c))
    return (ssum[0, 0] / jnp.maximum(cnt[0, 0], 1.0)).astype(output.dtype)
